# baseline (speedup 1.0000x reference)
import jax
import jax.numpy as jnp
from jax import lax
from jax.experimental import pallas as pl
from jax.experimental.pallas import tpu as pltpu
from jax.experimental.pallas import tpu_sc as plsc

LR_ = 0.01
WD_ = 1e-05
N_ITEMS = 1000000
N_USERS = 100000
DIM = 16
B_ = 16384

NCORE = 2
NSUB = 16
PER_TILE = B_ // NSUB
CHUNK = 128
NCHUNK = PER_TILE // CHUNK

ROWS_BLK = 4000
N_IBLK = N_ITEMS // ROWS_BLK
N_UBLK = N_USERS // ROWS_BLK
N_OUT = N_ITEMS + N_USERS


def _dense_body(item_ref, user_ref, out_ref):
    i = pl.program_id(0)

    @pl.when(i < N_IBLK)
    def _():
        out_ref[...] = item_ref[...] * (1.0 - WD_)

    @pl.when(i >= N_IBLK)
    def _():
        out_ref[...] = user_ref[...] * (1.0 - WD_)


def _dense_scale(item_emb, user_emb):
    return pl.pallas_call(
        _dense_body,
        grid=(N_IBLK + N_UBLK,),
        in_specs=[
            pl.BlockSpec((ROWS_BLK, DIM),
                         lambda i: (jnp.minimum(i, N_IBLK - 1), 0)),
            pl.BlockSpec((ROWS_BLK, DIM),
                         lambda i: (jnp.maximum(i - N_IBLK, 0), 0)),
        ],
        out_specs=pl.BlockSpec((ROWS_BLK, DIM), lambda i: (i, 0)),
        out_shape=jax.ShapeDtypeStruct((N_OUT, DIM), jnp.float32),
    )(item_emb, user_emb)


def _sc_body(out_hbm, idxg_hbm, idxl_hbm, occ_hbm, grads_hbm, zeros2_hbm,
             zeros1_hbm, ones1_hbm,
             slot_sp, cnt_sp, accum_sp,
             idxg_v, idxl_v, occ_v, rep_v, cnt_v, inv_v, ones_v,
             grad_v, base_v):
    c = lax.axis_index("c")
    s = lax.axis_index("s")
    occ_base = c * B_ + s * PER_TILE
    row8 = s * NCHUNK

    pltpu.sync_copy(idxg_hbm.at[pl.ds(row8 + c * (B_ // CHUNK), NCHUNK)], idxg_v)
    pltpu.sync_copy(idxl_hbm.at[pl.ds(row8 + c * (B_ // CHUNK), NCHUNK)], idxl_v)
    pltpu.sync_copy(occ_hbm.at[pl.ds(row8, NCHUNK)], occ_v)
    pltpu.sync_copy(grads_hbm.at[pl.ds(occ_base, PER_TILE)], grad_v)
    pltpu.sync_copy(ones1_hbm, ones_v)

    pltpu.sync_copy(zeros2_hbm, accum_sp.at[pl.ds(s * PER_TILE, PER_TILE)])
    pltpu.sync_copy(zeros1_hbm, cnt_sp.at[pl.ds(s * PER_TILE, PER_TILE)])

    for j in range(NCHUNK):
        pltpu.sync_copy(occ_v.at[j], slot_sp.at[idxl_v.at[j]])

    plsc.subcore_barrier()

    for j in range(NCHUNK):
        pltpu.sync_copy(slot_sp.at[idxl_v.at[j]], rep_v.at[j])

    for j in range(NCHUNK):
        pltpu.sync_copy(ones_v, cnt_sp.at[rep_v.at[j]], add=True)

    plsc.subcore_barrier()

    for j in range(NCHUNK):
        pltpu.sync_copy(cnt_sp.at[rep_v.at[j]], cnt_v.at[j])
    for j in range(NCHUNK):
        for k in range(CHUNK // 16):
            inv_v[j, pl.ds(k * 16, 16)] = LR_ / cnt_v[j, pl.ds(k * 16, 16)]

    for j in range(NCHUNK):
        @pl.loop(0, CHUNK // 16)
        def _(k, j=j):
            iv = inv_v[j, pl.ds(k * 16, 16)]
            base_i = j * CHUNK + k * 16
            for r in range(16):
                grad_v[base_i + r, :] = iv[r] * grad_v[base_i + r, :]

    for j in range(NCHUNK):
        pltpu.sync_copy(grad_v.at[pl.ds(j * CHUNK, CHUNK)],
                        accum_sp.at[rep_v.at[j]], add=True)
    for j in range(NCHUNK):
        pltpu.sync_copy(out_hbm.at[idxg_v.at[j]],
                        base_v.at[pl.ds(j * CHUNK, CHUNK)])

    plsc.subcore_barrier()

    for j in range(NCHUNK):
        pltpu.sync_copy(accum_sp.at[rep_v.at[j]],
                        grad_v.at[pl.ds(j * CHUNK, CHUNK)])

    @pl.loop(0, PER_TILE)
    def _(i):
        base_v[i, :] = base_v[i, :] - grad_v[i, :]

    for j in range(NCHUNK):
        pltpu.sync_copy(base_v.at[pl.ds(j * CHUNK, CHUNK)],
                        out_hbm.at[idxg_v.at[j]])


_sc_fixup = pl.kernel(
    _sc_body,
    out_type=(),
    mesh=plsc.VectorSubcoreMesh(core_axis_name="c", subcore_axis_name="s"),
    compiler_params=pltpu.CompilerParams(use_tc_tiling_on_sc=False),
    scratch_types=[
        pltpu.VMEM_SHARED((N_ITEMS,), jnp.int32),
        pltpu.VMEM_SHARED((B_,), jnp.float32),
        pltpu.VMEM_SHARED((B_, DIM), jnp.float32),
        pltpu.VMEM((NCHUNK, CHUNK), jnp.int32),
        pltpu.VMEM((NCHUNK, CHUNK), jnp.int32),
        pltpu.VMEM((NCHUNK, CHUNK), jnp.int32),
        pltpu.VMEM((NCHUNK, CHUNK), jnp.int32),
        pltpu.VMEM((NCHUNK, CHUNK), jnp.float32),
        pltpu.VMEM((NCHUNK, CHUNK), jnp.float32),
        pltpu.VMEM((CHUNK,), jnp.float32),
        pltpu.VMEM((PER_TILE, DIM), jnp.float32),
        pltpu.VMEM((PER_TILE, DIM), jnp.float32),
    ],
)


def kernel(item_emb, user_emb, item_grad, user_grad, returned_items,
           returned_users):
    ri = returned_items.astype(jnp.int32)
    ru = returned_users.astype(jnp.int32)
    idx_g = jnp.concatenate([ri, ru + N_ITEMS]).reshape(2 * B_ // CHUNK, CHUNK)
    idx_l = jnp.concatenate([ri, ru]).reshape(2 * B_ // CHUNK, CHUNK)
    occ = jnp.arange(B_, dtype=jnp.int32).reshape(B_ // CHUNK, CHUNK)
    grads = jnp.concatenate([item_grad, user_grad], axis=0)
    zeros2 = jnp.zeros((PER_TILE, DIM), jnp.float32)
    zeros1 = jnp.zeros((PER_TILE,), jnp.float32)
    ones1 = jnp.ones((CHUNK,), jnp.float32)

    dense = _dense_scale(item_emb, user_emb)
    out_ref = jax.new_ref(dense)
    _sc_fixup(out_ref, idx_g, idx_l, occ, grads, zeros2, zeros1, ones1)
    return out_ref[...]


# speedup vs baseline: 1.4475x; 1.4475x over previous
import jax
import jax.numpy as jnp
from jax import lax
from jax.experimental import pallas as pl
from jax.experimental.pallas import tpu as pltpu
from jax.experimental.pallas import tpu_sc as plsc

LR_ = 0.01
WD_ = 1e-05
N_ITEMS = 1000000
N_USERS = 100000
DIM = 16
B_ = 16384

NCORE = 2
NSUB = 16
PER_TILE = B_ // NSUB
CHUNK = 128
NCHUNK = PER_TILE // CHUNK

N_OUT = N_ITEMS + N_USERS
IT_P = N_ITEMS * DIM // 128
US_P = N_USERS * DIM // 128
IT_BLK = 5000


def _dense_body(in_ref, out_ref):
    out_ref[...] = in_ref[...] * (1.0 - WD_)


def _dense_scale(x128, blk):
    rows = x128.shape[0]
    return pl.pallas_call(
        _dense_body,
        grid=(rows // blk,),
        in_specs=[pl.BlockSpec((blk, 128), lambda i: (i, 0))],
        out_specs=pl.BlockSpec((blk, 128), lambda i: (i, 0)),
        out_shape=jax.ShapeDtypeStruct((rows, 128), jnp.float32),
    )(x128)


def _sc_body(ito_hbm, uso_hbm, idxl_hbm, occ_hbm, grads_hbm, zeros2_hbm,
             zeros1_hbm, ones1_hbm,
             slot_sp, cnt_sp, accum_sp,
             idxl_v, occ_v, rep_v, cnt_v, inv_v, ones_v,
             grad_v, base_v):
    c = lax.axis_index("c")
    s = lax.axis_index("s")
    occ_base = c * B_ + s * PER_TILE
    row8 = s * NCHUNK

    pltpu.sync_copy(idxl_hbm.at[pl.ds(row8 + c * (B_ // CHUNK), NCHUNK)], idxl_v)
    pltpu.sync_copy(occ_hbm.at[pl.ds(row8, NCHUNK)], occ_v)
    pltpu.sync_copy(grads_hbm.at[pl.ds(occ_base, PER_TILE)], grad_v)
    pltpu.sync_copy(ones1_hbm, ones_v)

    pltpu.sync_copy(zeros2_hbm, accum_sp.at[pl.ds(s * PER_TILE, PER_TILE)])
    pltpu.sync_copy(zeros1_hbm, cnt_sp.at[pl.ds(s * PER_TILE, PER_TILE)])

    for j in range(NCHUNK):
        pltpu.sync_copy(occ_v.at[j], slot_sp.at[idxl_v.at[j]])

    plsc.subcore_barrier()

    for j in range(NCHUNK):
        pltpu.sync_copy(slot_sp.at[idxl_v.at[j]], rep_v.at[j])

    for j in range(NCHUNK):
        pltpu.sync_copy(ones_v, cnt_sp.at[rep_v.at[j]], add=True)

    plsc.subcore_barrier()

    for j in range(NCHUNK):
        pltpu.sync_copy(cnt_sp.at[rep_v.at[j]], cnt_v.at[j])
    for j in range(NCHUNK):
        for k in range(CHUNK // 16):
            inv_v[j, pl.ds(k * 16, 16)] = LR_ / cnt_v[j, pl.ds(k * 16, 16)]

    for j in range(NCHUNK):
        @pl.loop(0, CHUNK // 16)
        def _(k, j=j):
            iv = inv_v[j, pl.ds(k * 16, 16)]
            base_i = j * CHUNK + k * 16
            for r in range(16):
                grad_v[base_i + r, :] = iv[r] * grad_v[base_i + r, :]

    for j in range(NCHUNK):
        pltpu.sync_copy(grad_v.at[pl.ds(j * CHUNK, CHUNK)],
                        accum_sp.at[rep_v.at[j]], add=True)
    @pl.when(c == 0)
    def _():
        for j in range(NCHUNK):
            pltpu.sync_copy(ito_hbm.at[idxl_v.at[j]],
                            base_v.at[pl.ds(j * CHUNK, CHUNK)])

    @pl.when(c == 1)
    def _():
        for j in range(NCHUNK):
            pltpu.sync_copy(uso_hbm.at[idxl_v.at[j]],
                            base_v.at[pl.ds(j * CHUNK, CHUNK)])

    plsc.subcore_barrier()

    for j in range(NCHUNK):
        pltpu.sync_copy(accum_sp.at[rep_v.at[j]],
                        grad_v.at[pl.ds(j * CHUNK, CHUNK)])

    @pl.loop(0, PER_TILE)
    def _(i):
        base_v[i, :] = base_v[i, :] - grad_v[i, :]

    @pl.when(c == 0)
    def _():
        for j in range(NCHUNK):
            pltpu.sync_copy(base_v.at[pl.ds(j * CHUNK, CHUNK)],
                            ito_hbm.at[idxl_v.at[j]])

    @pl.when(c == 1)
    def _():
        for j in range(NCHUNK):
            pltpu.sync_copy(base_v.at[pl.ds(j * CHUNK, CHUNK)],
                            uso_hbm.at[idxl_v.at[j]])


_sc_fixup = pl.kernel(
    _sc_body,
    out_type=(),
    mesh=plsc.VectorSubcoreMesh(core_axis_name="c", subcore_axis_name="s"),
    compiler_params=pltpu.CompilerParams(use_tc_tiling_on_sc=False),
    scratch_types=[
        pltpu.VMEM_SHARED((N_ITEMS,), jnp.int32),
        pltpu.VMEM_SHARED((B_,), jnp.float32),
        pltpu.VMEM_SHARED((B_, DIM), jnp.float32),
        pltpu.VMEM((NCHUNK, CHUNK), jnp.int32),
        pltpu.VMEM((NCHUNK, CHUNK), jnp.int32),
        pltpu.VMEM((NCHUNK, CHUNK), jnp.int32),
        pltpu.VMEM((NCHUNK, CHUNK), jnp.float32),
        pltpu.VMEM((NCHUNK, CHUNK), jnp.float32),
        pltpu.VMEM((CHUNK,), jnp.float32),
        pltpu.VMEM((PER_TILE, DIM), jnp.float32),
        pltpu.VMEM((PER_TILE, DIM), jnp.float32),
    ],
)


def kernel(item_emb, user_emb, item_grad, user_grad, returned_items,
           returned_users):
    ri = returned_items.astype(jnp.int32)
    ru = returned_users.astype(jnp.int32)
    idx_l = jnp.concatenate([ri, ru]).reshape(2 * B_ // CHUNK, CHUNK)
    occ = jnp.arange(B_, dtype=jnp.int32).reshape(B_ // CHUNK, CHUNK)
    grads = jnp.concatenate([item_grad, user_grad], axis=0)
    zeros2 = jnp.zeros((PER_TILE, DIM), jnp.float32)
    zeros1 = jnp.zeros((PER_TILE,), jnp.float32)
    ones1 = jnp.ones((CHUNK,), jnp.float32)

    ito = _dense_scale(item_emb.reshape(IT_P, 128), IT_BLK).reshape(N_ITEMS, DIM)
    uso = _dense_scale(user_emb.reshape(US_P, 128), US_P).reshape(N_USERS, DIM)
    ito_ref = jax.new_ref(ito)
    uso_ref = jax.new_ref(uso)
    _sc_fixup(ito_ref, uso_ref, idx_l, occ, grads, zeros2, zeros1, ones1)
    return jnp.concatenate([ito_ref[...], uso_ref[...]], axis=0)


# speedup vs baseline: 2.1641x; 1.4951x over previous
import jax
import jax.numpy as jnp
from jax import lax
from jax.experimental import pallas as pl
from jax.experimental.pallas import tpu as pltpu
from jax.experimental.pallas import tpu_sc as plsc

LR_ = 0.01
WD_ = 1e-05
N_ITEMS = 1000000
N_USERS = 100000
DIM = 16
B_ = 16384

NCORE = 2
NSUB = 16
PER_TILE = B_ // NSUB
CHUNK = 128
NCHUNK = PER_TILE // CHUNK

N_OUT = N_ITEMS + N_USERS
CB = 8192


def _combine_body(tin_ref, d_ref, out_ref):
    out_ref[...] = tin_ref[...] * (1.0 - WD_) - d_ref[...].T


def _combine(tin, delta):
    n = tin.shape[1]
    grid = (n + CB - 1) // CB
    return pl.pallas_call(
        _combine_body,
        grid=(grid,),
        in_specs=[
            pl.BlockSpec((16, CB), lambda i: (0, i)),
            pl.BlockSpec((CB, DIM), lambda i: (i, 0)),
        ],
        out_specs=pl.BlockSpec((16, CB), lambda i: (0, i)),
        out_shape=jax.ShapeDtypeStruct((16, n), jnp.float32),
    )(tin, delta)


def _sc_body(di_hbm, du_hbm, idxl_hbm, occ_hbm, grads_hbm, zeros2_hbm,
             zeros1_hbm, ones1_hbm,
             slot_sp, cnt_sp, accum_sp,
             idxl_v, occ_v, rep_v, cnt_v, inv_v, ones_v,
             grad_v):
    c = lax.axis_index("c")
    s = lax.axis_index("s")
    occ_base = c * B_ + s * PER_TILE
    row8 = s * NCHUNK

    pltpu.sync_copy(idxl_hbm.at[pl.ds(row8 + c * (B_ // CHUNK), NCHUNK)], idxl_v)
    pltpu.sync_copy(occ_hbm.at[pl.ds(row8, NCHUNK)], occ_v)
    pltpu.sync_copy(grads_hbm.at[pl.ds(occ_base, PER_TILE)], grad_v)
    pltpu.sync_copy(ones1_hbm, ones_v)

    pltpu.sync_copy(zeros2_hbm, accum_sp.at[pl.ds(s * PER_TILE, PER_TILE)])
    pltpu.sync_copy(zeros1_hbm, cnt_sp.at[pl.ds(s * PER_TILE, PER_TILE)])

    for j in range(NCHUNK):
        pltpu.sync_copy(occ_v.at[j], slot_sp.at[idxl_v.at[j]])

    plsc.subcore_barrier()

    for j in range(NCHUNK):
        pltpu.sync_copy(slot_sp.at[idxl_v.at[j]], rep_v.at[j])

    for j in range(NCHUNK):
        pltpu.sync_copy(ones_v, cnt_sp.at[rep_v.at[j]], add=True)

    plsc.subcore_barrier()

    for j in range(NCHUNK):
        pltpu.sync_copy(cnt_sp.at[rep_v.at[j]], cnt_v.at[j])
    for j in range(NCHUNK):
        for k in range(CHUNK // 16):
            inv_v[j, pl.ds(k * 16, 16)] = LR_ / cnt_v[j, pl.ds(k * 16, 16)]

    for j in range(NCHUNK):
        @pl.loop(0, CHUNK // 16)
        def _(k, j=j):
            iv = inv_v[j, pl.ds(k * 16, 16)]
            base_i = j * CHUNK + k * 16
            for r in range(16):
                grad_v[base_i + r, :] = iv[r] * grad_v[base_i + r, :]

    for j in range(NCHUNK):
        pltpu.sync_copy(grad_v.at[pl.ds(j * CHUNK, CHUNK)],
                        accum_sp.at[rep_v.at[j]], add=True)

    plsc.subcore_barrier()

    for j in range(NCHUNK):
        pltpu.sync_copy(accum_sp.at[rep_v.at[j]],
                        grad_v.at[pl.ds(j * CHUNK, CHUNK)])

    @pl.when(c == 0)
    def _():
        for j in range(NCHUNK):
            pltpu.sync_copy(grad_v.at[pl.ds(j * CHUNK, CHUNK)],
                            di_hbm.at[idxl_v.at[j]])

    @pl.when(c == 1)
    def _():
        for j in range(NCHUNK):
            pltpu.sync_copy(grad_v.at[pl.ds(j * CHUNK, CHUNK)],
                            du_hbm.at[idxl_v.at[j]])


_sc_fixup = pl.kernel(
    _sc_body,
    out_type=(),
    mesh=plsc.VectorSubcoreMesh(core_axis_name="c", subcore_axis_name="s"),
    compiler_params=pltpu.CompilerParams(use_tc_tiling_on_sc=False),
    scratch_types=[
        pltpu.VMEM_SHARED((N_ITEMS,), jnp.int32),
        pltpu.VMEM_SHARED((B_,), jnp.float32),
        pltpu.VMEM_SHARED((B_, DIM), jnp.float32),
        pltpu.VMEM((NCHUNK, CHUNK), jnp.int32),
        pltpu.VMEM((NCHUNK, CHUNK), jnp.int32),
        pltpu.VMEM((NCHUNK, CHUNK), jnp.int32),
        pltpu.VMEM((NCHUNK, CHUNK), jnp.float32),
        pltpu.VMEM((NCHUNK, CHUNK), jnp.float32),
        pltpu.VMEM((CHUNK,), jnp.float32),
        pltpu.VMEM((PER_TILE, DIM), jnp.float32),
    ],
)


def kernel(item_emb, user_emb, item_grad, user_grad, returned_items,
           returned_users):
    ri = returned_items.astype(jnp.int32)
    ru = returned_users.astype(jnp.int32)
    idx_l = jnp.concatenate([ri, ru]).reshape(2 * B_ // CHUNK, CHUNK)
    occ = jnp.arange(B_, dtype=jnp.int32).reshape(B_ // CHUNK, CHUNK)
    grads = jnp.concatenate([item_grad, user_grad], axis=0)
    zeros2 = jnp.zeros((PER_TILE, DIM), jnp.float32)
    zeros1 = jnp.zeros((PER_TILE,), jnp.float32)
    ones1 = jnp.ones((CHUNK,), jnp.float32)

    di_ref = jax.new_ref(jnp.zeros((N_ITEMS, DIM), jnp.float32))
    du_ref = jax.new_ref(jnp.zeros((N_USERS, DIM), jnp.float32))
    _sc_fixup(di_ref, du_ref, idx_l, occ, grads, zeros2, zeros1, ones1)
    out_i = _combine(item_emb.T, di_ref[...])
    out_u = _combine(user_emb.T, du_ref[...])
    return jnp.concatenate([out_i.T, out_u.T], axis=0)


# speedup vs baseline: 2.1938x; 1.0137x over previous
import jax
import jax.numpy as jnp
from jax import lax
from jax.experimental import pallas as pl
from jax.experimental.pallas import tpu as pltpu
from jax.experimental.pallas import tpu_sc as plsc

LR_ = 0.01
WD_ = 1e-05
N_ITEMS = 1000000
N_USERS = 100000
DIM = 16
B_ = 16384

NCORE = 2
NSUB = 16
PER_TILE = B_ // NSUB
CHUNK = 128
NCHUNK = PER_TILE // CHUNK

N_OUT = N_ITEMS + N_USERS
CB = 8192


def _combine_body(tin_ref, d_ref, out_ref):
    out_ref[...] = tin_ref[...] * (1.0 - WD_) - d_ref[...]


def _combine(tin, delta_t):
    n = tin.shape[1]
    grid = (n + CB - 1) // CB
    return pl.pallas_call(
        _combine_body,
        grid=(grid,),
        in_specs=[
            pl.BlockSpec((16, CB), lambda i: (0, i)),
            pl.BlockSpec((16, CB), lambda i: (0, i)),
        ],
        out_specs=pl.BlockSpec((16, CB), lambda i: (0, i)),
        out_shape=jax.ShapeDtypeStruct((16, n), jnp.float32),
    )(tin, delta_t)


def _sc_body(di_hbm, du_hbm, idxl_hbm, occ_hbm, grads_hbm, zeros2_hbm,
             zeros1_hbm, ones1_hbm,
             slot_sp, cnt_sp, accum_sp,
             idxl_v, occ_v, rep_v, cnt_v, inv_v, ones_v,
             grad_v):
    c = lax.axis_index("c")
    s = lax.axis_index("s")
    occ_base = c * B_ + s * PER_TILE
    row8 = s * NCHUNK

    pltpu.sync_copy(idxl_hbm.at[pl.ds(row8 + c * (B_ // CHUNK), NCHUNK)], idxl_v)
    pltpu.sync_copy(occ_hbm.at[pl.ds(row8, NCHUNK)], occ_v)
    pltpu.sync_copy(grads_hbm.at[pl.ds(occ_base, PER_TILE)], grad_v)
    pltpu.sync_copy(ones1_hbm, ones_v)

    pltpu.sync_copy(zeros2_hbm, accum_sp.at[pl.ds(s * PER_TILE, PER_TILE)])
    pltpu.sync_copy(zeros1_hbm, cnt_sp.at[pl.ds(s * PER_TILE, PER_TILE)])

    for j in range(NCHUNK):
        pltpu.sync_copy(occ_v.at[j], slot_sp.at[idxl_v.at[j]])

    plsc.subcore_barrier()

    for j in range(NCHUNK):
        pltpu.sync_copy(slot_sp.at[idxl_v.at[j]], rep_v.at[j])

    for j in range(NCHUNK):
        pltpu.sync_copy(ones_v, cnt_sp.at[rep_v.at[j]], add=True)

    plsc.subcore_barrier()

    for j in range(NCHUNK):
        pltpu.sync_copy(cnt_sp.at[rep_v.at[j]], cnt_v.at[j])
    for j in range(NCHUNK):
        for k in range(CHUNK // 16):
            inv_v[j, pl.ds(k * 16, 16)] = LR_ / cnt_v[j, pl.ds(k * 16, 16)]

    for j in range(NCHUNK):
        @pl.loop(0, CHUNK // 16)
        def _(k, j=j):
            iv = inv_v[j, pl.ds(k * 16, 16)]
            base_i = j * CHUNK + k * 16
            for r in range(16):
                grad_v[base_i + r, :] = iv[r] * grad_v[base_i + r, :]

    for j in range(NCHUNK):
        pltpu.sync_copy(grad_v.at[pl.ds(j * CHUNK, CHUNK)],
                        accum_sp.at[rep_v.at[j]], add=True)

    plsc.subcore_barrier()

    for j in range(NCHUNK):
        pltpu.sync_copy(accum_sp.at[rep_v.at[j]],
                        grad_v.at[pl.ds(j * CHUNK, CHUNK)])

    @pl.when(c == 0)
    def _():
        for j in range(NCHUNK):
            pltpu.sync_copy(grad_v.at[pl.ds(j * CHUNK, CHUNK)],
                            di_hbm.at[idxl_v.at[j]])

    @pl.when(c == 1)
    def _():
        for j in range(NCHUNK):
            pltpu.sync_copy(grad_v.at[pl.ds(j * CHUNK, CHUNK)],
                            du_hbm.at[idxl_v.at[j]])


_sc_fixup = pl.kernel(
    _sc_body,
    out_type=(),
    mesh=plsc.VectorSubcoreMesh(core_axis_name="c", subcore_axis_name="s"),
    compiler_params=pltpu.CompilerParams(use_tc_tiling_on_sc=False),
    scratch_types=[
        pltpu.VMEM_SHARED((N_ITEMS,), jnp.int32),
        pltpu.VMEM_SHARED((B_,), jnp.float32),
        pltpu.VMEM_SHARED((B_, DIM), jnp.float32),
        pltpu.VMEM((NCHUNK, CHUNK), jnp.int32),
        pltpu.VMEM((NCHUNK, CHUNK), jnp.int32),
        pltpu.VMEM((NCHUNK, CHUNK), jnp.int32),
        pltpu.VMEM((NCHUNK, CHUNK), jnp.float32),
        pltpu.VMEM((NCHUNK, CHUNK), jnp.float32),
        pltpu.VMEM((CHUNK,), jnp.float32),
        pltpu.VMEM((PER_TILE, DIM), jnp.float32),
    ],
)


def kernel(item_emb, user_emb, item_grad, user_grad, returned_items,
           returned_users):
    ri = returned_items.astype(jnp.int32)
    ru = returned_users.astype(jnp.int32)
    idx_l = jnp.concatenate([ri, ru]).reshape(2 * B_ // CHUNK, CHUNK)
    occ = jnp.arange(B_, dtype=jnp.int32).reshape(B_ // CHUNK, CHUNK)
    grads = jnp.concatenate([item_grad, user_grad], axis=0)
    zeros2 = jnp.zeros((PER_TILE, DIM), jnp.float32)
    zeros1 = jnp.zeros((PER_TILE,), jnp.float32)
    ones1 = jnp.ones((CHUNK,), jnp.float32)

    di_ref = jax.new_ref(jnp.zeros((N_ITEMS, DIM), jnp.float32))
    du_ref = jax.new_ref(jnp.zeros((N_USERS, DIM), jnp.float32))
    _sc_fixup(di_ref, du_ref, idx_l, occ, grads, zeros2, zeros1, ones1)
    out_i = _combine(item_emb.T, di_ref[...].T)
    out_u = _combine(user_emb.T, du_ref[...].T)
    return jnp.concatenate([out_i.T, out_u.T], axis=0)
